# baseline (speedup 1.0000x reference)
import jax
import jax.numpy as jnp
from jax.experimental import pallas as pl

B, N, K = 4, 4096, 16
RBLK = 256


def _dist_body(pts_ref, ptsT_ref, d_ref):
    xi = pts_ref[0, :, 0:1]
    yi = pts_ref[0, :, 1:2]
    zi = pts_ref[0, :, 2:3]
    xj = ptsT_ref[0, 0:1, :]
    yj = ptsT_ref[0, 1:2, :]
    zj = ptsT_ref[0, 2:3, :]
    bf = jnp.bfloat16
    f32 = jnp.float32
    xbi, ybi, zbi = (c.astype(bf).astype(f32) for c in (xi, yi, zi))
    xbj, ybj, zbj = (c.astype(bf).astype(f32) for c in (xj, yj, zj))
    dot = xbi * xbj + ybi * ybj + zbi * zbj
    sqi = xi * xi + yi * yi + zi * zi
    sqj = xj * xj + yj * yj + zj * zj
    d_ref[0, :, :] = (sqi + sqj) - 2.0 * dot


def _pairwise_sq_dists(points):
    ptsT = jnp.transpose(points, (0, 2, 1))
    return pl.pallas_call(
        _dist_body,
        grid=(B, N // RBLK),
        in_specs=[
            pl.BlockSpec((1, RBLK, 3), lambda b, i: (b, i, 0)),
            pl.BlockSpec((1, 3, N), lambda b, i: (b, 0, 0)),
        ],
        out_specs=pl.BlockSpec((1, RBLK, N), lambda b, i: (b, i, 0)),
        out_shape=jax.ShapeDtypeStruct((B, N, N), jnp.float32),
    )(points, ptsT)


def kernel(points):
    d = _pairwise_sq_dists(points)
    _, idx = jax.lax.top_k(-d, K)
    return idx


# speedup vs baseline: 5.5269x; 5.5269x over previous
import dataclasses
import functools

import jax
import jax.numpy as jnp
from jax import lax
from jax.experimental import pallas as pl
from jax.experimental.pallas import tpu as pltpu
from jax.experimental.pallas import tpu_sc as plsc

B, N, K = 4, 4096, 16
RBLK = 256
NW = 32
CH = 8
STEPS = (B * N) // (NW * CH)
L = 16
FMAX = 3.4028235e38


def _dist_body(pts_ref, ptsT_ref, d_ref):
    xi = pts_ref[0, :, 0:1]
    yi = pts_ref[0, :, 1:2]
    zi = pts_ref[0, :, 2:3]
    xj = ptsT_ref[0, 0:1, :]
    yj = ptsT_ref[0, 1:2, :]
    zj = ptsT_ref[0, 2:3, :]
    bf = jnp.bfloat16
    f32 = jnp.float32
    xbi, ybi, zbi = (c.astype(bf).astype(f32) for c in (xi, yi, zi))
    xbj, ybj, zbj = (c.astype(bf).astype(f32) for c in (xj, yj, zj))
    dot = xbi * xbj + ybi * ybj + zbi * zbj
    sqi = xi * xi + yi * yi + zi * zi
    sqj = xj * xj + yj * yj + zj * zj
    d_ref[0, :, :] = (sqi + sqj) - 2.0 * dot


def _pairwise_sq_dists(points):
    ptsT = jnp.transpose(points, (0, 2, 1))
    return pl.pallas_call(
        _dist_body,
        grid=(B, N // RBLK),
        in_specs=[
            pl.BlockSpec((1, RBLK, 3), lambda b, i: (b, i, 0)),
            pl.BlockSpec((1, 3, N), lambda b, i: (b, 0, 0)),
        ],
        out_specs=pl.BlockSpec((1, RBLK, N), lambda b, i: (b, i, 0)),
        out_shape=jax.ShapeDtypeStruct((B, N, N), jnp.float32),
    )(points, ptsT)


def _topk_row(d_vmem, cand_d, cand_j, w_ref, out_vmem, r):
    iota = lax.iota(jnp.int32, L)

    def p1_body(v, carry):
        m0, m1, m2, m3 = carry
        base = v * (4 * L)
        m0 = jnp.minimum(m0, d_vmem[r, pl.ds(base, L)])
        m1 = jnp.minimum(m1, d_vmem[r, pl.ds(base + L, L)])
        m2 = jnp.minimum(m2, d_vmem[r, pl.ds(base + 2 * L, L)])
        m3 = jnp.minimum(m3, d_vmem[r, pl.ds(base + 3 * L, L)])
        return m0, m1, m2, m3

    inf_v = jnp.full((L,), FMAX, jnp.float32)
    m0, m1, m2, m3 = lax.fori_loop(
        0, N // (4 * L), p1_body, (inf_v, inf_v, inf_v, inf_v))
    tau = jnp.max(jnp.minimum(jnp.minimum(m0, m1), jnp.minimum(m2, m3)))
    tau_v = jnp.full((L,), tau, jnp.float32)

    w_ref[0] = 0

    @pl.loop(0, N, step=4 * L)
    def p2_body(c):
        ds = [d_vmem[r, pl.ds(c + p * L, L)] for p in range(4)]
        les = [d <= tau_v for d in ds]
        hit = jnp.any((les[0] | les[1]) | (les[2] | les[3]))

        @pl.when(hit)
        def _():
            for p in range(4):
                le = les[p]

                @pl.when(jnp.any(le))
                def _(p=p, le=le):
                    cum = plsc.cumsum(le.astype(jnp.int32))
                    pos = (w_ref[0] - 1) + cum
                    jvec = iota + (c + p * L)
                    plsc.store_scatter(cand_d, [pos], ds[p], mask=le)
                    plsc.store_scatter(cand_j, [pos], jvec, mask=le)
                    w_ref[0] = w_ref[0] + jnp.sum(le.astype(jnp.int32))

    w = w_ref[0]
    w_v = jnp.full((L,), w, jnp.int32)
    nchunks = (w + (L - 1)) // L

    def p3_body(t, carry):
        bd, bj = carry
        cd = cand_d[pl.ds(t * L, L)]
        cj = cand_j[pl.ds(t * L, L)]
        valid = (iota + t * L) < w_v
        cd = jnp.where(valid, cd, FMAX)
        scd, scj = plsc.sort_key_val(cd, cj)
        rb = lax.rev(bd, (0,))
        rbj = lax.rev(bj, (0,))
        take = scd <= rb
        nd = jnp.where(take, scd, rb)
        nj = jnp.where(take, scj, rbj)
        bd, bj = plsc.sort_key_val(nd, nj)
        return bd, bj

    bd, bj = lax.fori_loop(0, nchunks, p3_body,
                           (inf_v, jnp.zeros((L,), jnp.int32)))
    out_vmem[r, :] = bj


def _sc_topk(d):
    mesh = plsc.VectorSubcoreMesh(core_axis_name="core",
                                  subcore_axis_name="subcore")
    cp = pltpu.CompilerParams()
    if "needs_layout_passes" in pltpu.CompilerParams.__dataclass_fields__:
        cp = dataclasses.replace(cp, needs_layout_passes=False)

    @functools.partial(
        pl.kernel,
        compiler_params=cp,
        out_type=jax.ShapeDtypeStruct((B * N, K), jnp.int32),
        mesh=mesh,
        scratch_types=[
            pltpu.VMEM((N,), jnp.float32),
            pltpu.VMEM((N,), jnp.int32),
            pltpu.SMEM((1,), jnp.int32),
        ],
    )
    def sc_kernel(d_hbm, out_hbm, cand_d, cand_j, w_ref):
        def body(d_vmem, out_vmem):
            for r in range(CH):
                _topk_row(d_vmem, cand_d, cand_j, w_ref, out_vmem, r)

        pltpu.emit_pipeline(
            body,
            grid=(NW, STEPS),
            in_specs=[pl.BlockSpec((CH, N), lambda i, j: (i * STEPS + j, 0))],
            out_specs=[pl.BlockSpec((CH, K), lambda i, j: (i * STEPS + j, 0))],
            core_axis_name=("core", "subcore"),
            dimension_semantics=(pltpu.PARALLEL, pltpu.ARBITRARY),
        )(d_hbm, out_hbm)

    return sc_kernel(d)


def kernel(points):
    d = _pairwise_sq_dists(points)
    idx = _sc_topk(d.reshape(B * N, N))
    return idx.reshape(B, N, K)
